# baseline (speedup 1.0000x reference)
import functools

import jax
import jax.numpy as jnp
from jax import lax
from jax.experimental import pallas as pl
from jax.experimental.pallas import tpu as pltpu

_B = 4
_N = 8192
_P = 1024
_K = 32
_R2 = 0.2 ** 2
_OUT = 64
_TP = 64
_MROWS = 2048


def _fps_body(px_ref, py_ref, pz_ref, cx_ref, cy_ref, cz_ref, dist_ref):
    px = px_ref[...]
    py = py_ref[...]
    pz = pz_ref[...]
    lane = lax.broadcasted_iota(jnp.int32, (_B, _N), 1)
    plane = lax.broadcasted_iota(jnp.int32, (_B, _P), 1)

    dx = px - px[:, 0:1]
    dy = py - py[:, 0:1]
    dz = pz - pz[:, 0:1]
    dist_ref[...] = (dx * dx + dy * dy) + dz * dz
    zmask = plane == 0
    cx_ref[...] = jnp.where(zmask, px[:, 0:1], 0.0)
    cy_ref[...] = jnp.where(zmask, py[:, 0:1], 0.0)
    cz_ref[...] = jnp.where(zmask, pz[:, 0:1], 0.0)

    def body(i, _):
        dist = dist_ref[...]
        maxv = jnp.max(dist, axis=1, keepdims=True)
        idx = jnp.min(jnp.where(dist == maxv, lane, _N), axis=1, keepdims=True)
        oh = lane == idx
        cx = jnp.sum(jnp.where(oh, px, 0.0), axis=1, keepdims=True)
        cy = jnp.sum(jnp.where(oh, py, 0.0), axis=1, keepdims=True)
        cz = jnp.sum(jnp.where(oh, pz, 0.0), axis=1, keepdims=True)
        ddx = px - cx
        ddy = py - cy
        ddz = pz - cz
        d = (ddx * ddx + ddy * ddy) + ddz * ddz
        dist_ref[...] = jnp.minimum(dist, d)
        sm = plane == i
        cx_ref[...] += jnp.where(sm, cx, 0.0)
        cy_ref[...] += jnp.where(sm, cy, 0.0)
        cz_ref[...] += jnp.where(sm, cz, 0.0)
        return 0

    lax.fori_loop(1, _P, body, 0)


def _mlp_body(x_ref, w1_ref, b1_ref, w2_ref, b2_ref, o_ref):
    i = pl.program_id(0)
    o_ref[...] = jnp.zeros_like(o_ref)

    @pl.when(i < _B * _N // _MROWS)
    def _():
        x = x_ref[...]
        a = jnp.dot(x, w1_ref[...], preferred_element_type=jnp.float32)
        a = jnp.maximum(a + b1_ref[...], 0.0)
        o = jnp.dot(a, w2_ref[...], preferred_element_type=jnp.float32)
        o_ref[:, :_OUT] = jnp.maximum(o + b2_ref[...], 0.0)


_SENT = _B * _N


_NB = _N // 128
_LB = 128
_LV = 8


def _select_body(c8_ref, c2_ref, pT_ref, p2_ref, col_ref, sg_ref):
    b = pl.program_id(0)
    c8 = c8_ref[0]
    pT = pT_ref[0]
    cp = jnp.dot(c8, pT, preferred_element_type=jnp.float32)
    d2 = (c2_ref[0, 0][:, None] + p2_ref[0, 0][None, :]) - 2.0 * cp
    inf = jnp.float32(jnp.inf)

    key3 = jnp.where(d2 <= _R2, d2, inf).reshape(_TP, _NB, _LB)
    lane3 = lax.broadcasted_iota(jnp.int32, (_TP, _NB, _LB), 2)
    blk_base = lax.broadcasted_iota(jnp.int32, (_TP, _NB), 1) * _LB
    h_v, h_i = [], []
    for t in range(_LV):
        bm = jnp.min(key3, axis=2)
        eq = key3 == bm[:, :, None]
        bi = jnp.min(jnp.where(eq, lane3, _LB), axis=2)
        key3 = jnp.where(eq & (lane3 == bi[:, :, None]), inf, key3)
        h_v.append(bm)
        h_i.append(blk_base + bi)
    rem = jnp.min(key3, axis=2)
    has_more = rem < inf

    fallback = None
    exb = jnp.full((_TP, 1), inf, dtype=jnp.float32)
    flag = jnp.zeros((_TP,), dtype=jnp.bool_)
    for k in range(_K):
        gm = jnp.min(h_v[0], axis=1, keepdims=True)
        flag = flag | ((exb[:, 0] < inf) & (gm[:, 0] >= exb[:, 0]))
        valid = gm[:, 0] < inf
        gi = jnp.min(jnp.where(h_v[0] == gm, h_i[0], _N), axis=1)
        g = gi + b * _N
        if k == 0:
            fallback = jnp.where(valid, g, _SENT)
        col_ref[0, k, :] = jnp.where(valid, g, -1)
        sg_ref[0, k, :] = jnp.where(valid, g, fallback)
        pm = (h_v[0] == gm) & (h_i[0] == gi[:, None])
        for t in range(_LV - 1):
            h_v[t] = jnp.where(pm, h_v[t + 1], h_v[t])
            h_i[t] = jnp.where(pm, h_i[t + 1], h_i[t])
        h_v[_LV - 1] = jnp.where(pm, inf, h_v[_LV - 1])
        if k >= _LV - 1:
            exhausted = (h_v[0] == inf) & has_more
            exb = jnp.min(jnp.where(exhausted, rem, inf), axis=1,
                          keepdims=True)

    @pl.when(jnp.any(flag))
    def _():
        cp2 = jnp.dot(c8_ref[0], pT_ref[0], preferred_element_type=jnp.float32)
        dd2 = (c2_ref[0, 0][:, None] + p2_ref[0, 0][None, :]) - 2.0 * cp2
        key = jnp.where(dd2 <= _R2, dd2, inf)
        lane = lax.broadcasted_iota(jnp.int32, (_TP, _N), 1)
        fb = None
        for k in range(_K):
            minv = jnp.min(key, axis=1, keepdims=True)
            valid = minv[:, 0] < inf
            idx = jnp.min(jnp.where(key == minv, lane, _N), axis=1)
            g = idx + b * _N
            if k == 0:
                fb = jnp.where(valid, g, _SENT)
            col_ref[0, k, :] = jnp.where(valid, g, -1)
            sg_ref[0, k, :] = jnp.where(valid, g, fb)
            key = jnp.where(lane == idx[:, None], inf, key)


_NC = 2
_NS = 16
_NW = _NC * _NS
_CPW = _B * _P // _NW
_CCH = 4
_NCH = _CPW // _CCH
_ROWS = _CCH * _K


def _sc_gather_max(table, gidx3):
    from jax.experimental.pallas import tpu_sc as plsc

    mesh = plsc.VectorSubcoreMesh(core_axis_name="c", subcore_axis_name="s")

    @functools.partial(
        pl.kernel,
        mesh=mesh,
        out_type=jax.ShapeDtypeStruct((_B * _P, _OUT), jnp.float32),
        scratch_types=[
            pltpu.VMEM((_NCH, _ROWS), jnp.int32),
            pltpu.VMEM((_ROWS, 128), jnp.float32),
            pltpu.VMEM((_CPW, _OUT), jnp.float32),
            pltpu.SemaphoreType.DMA,
        ],
    )
    def k(table_hbm, gidx_hbm, out_hbm, idx_v, rows_v, out_v, sem):
        wid = lax.axis_index("s") * _NC + lax.axis_index("c")
        pltpu.sync_copy(gidx_hbm.at[wid], idx_v)

        def chunk(ci, _):
            pltpu.async_copy(table_hbm.at[idx_v.at[ci]], rows_v, sem).wait()

            def cent(j, _):
                base = j * _K
                for g in range(_OUT // 16):
                    sl = pl.ds(g * 16, 16)
                    acc = rows_v[base, sl]
                    for r in range(1, _K):
                        acc = jnp.maximum(acc, rows_v[base + r, sl])
                    out_v[ci * _CCH + j, sl] = acc
                return 0

            lax.fori_loop(0, _CCH, cent, 0)
            return 0

        lax.fori_loop(0, _NCH, chunk, 0)
        pltpu.sync_copy(out_v, out_hbm.at[pl.ds(wid * _CPW, _CPW)])

    return k(table, gidx3)


def kernel(pos, h, batch_indices, W1, b1, W2, b2):
    del batch_indices
    posB = pos.reshape(_B, _N, 3)
    px = posB[:, :, 0]
    py = posB[:, :, 1]
    pz = posB[:, :, 2]

    cx, cy, cz = pl.pallas_call(
        _fps_body,
        out_shape=[jax.ShapeDtypeStruct((_B, _P), jnp.float32)] * 3,
        scratch_shapes=[pltpu.VMEM((_B, _N), jnp.float32)],
    )(px, py, pz)
    centroids = jnp.stack([cx, cy, cz], axis=-1)

    hB = h.reshape(_B, _N, -1)
    feat = jnp.concatenate([hB, posB], axis=-1).reshape(_B * _N, 6)
    featp = jnp.concatenate(
        [feat, jnp.zeros((_B * _N, 2), jnp.float32)], axis=-1)
    W1p = jnp.concatenate([W1, jnp.zeros((2, _OUT), W1.dtype)], axis=0)
    nblk = _B * _N // _MROWS
    table = pl.pallas_call(
        _mlp_body,
        grid=(nblk + 1,),
        in_specs=[
            pl.BlockSpec((_MROWS, 8), lambda i: (jnp.minimum(i, nblk - 1), 0)),
            pl.BlockSpec((8, _OUT), lambda i: (0, 0)),
            pl.BlockSpec((1, _OUT), lambda i: (0, 0)),
            pl.BlockSpec((_OUT, _OUT), lambda i: (0, 0)),
            pl.BlockSpec((1, _OUT), lambda i: (0, 0)),
        ],
        out_specs=pl.BlockSpec((_MROWS, 128), lambda i: (i, 0)),
        out_shape=jax.ShapeDtypeStruct((_B * _N + _MROWS, 128), jnp.float32),
    )(featp, W1p, b1[None, :], W2, b2[None, :])

    c8 = jnp.concatenate(
        [centroids, jnp.zeros((_B, _P, 5), jnp.float32)], axis=-1)
    c2 = jnp.sum(centroids ** 2, -1)
    p2 = jnp.sum(posB ** 2, -1)
    pT = jnp.moveaxis(posB, 2, 1)
    pT8 = jnp.concatenate([pT, jnp.zeros((_B, 5, _N), jnp.float32)], axis=1)

    nt = _P // _TP
    colT, sgT = pl.pallas_call(
        _select_body,
        grid=(_B, nt),
        in_specs=[
            pl.BlockSpec((1, _TP, 8), lambda b, t: (b, t, 0)),
            pl.BlockSpec((1, 1, _TP), lambda b, t: (b * nt + t, 0, 0)),
            pl.BlockSpec((1, 8, _N), lambda b, t: (b, 0, 0)),
            pl.BlockSpec((1, 1, _N), lambda b, t: (b, 0, 0)),
        ],
        out_specs=[
            pl.BlockSpec((1, _K, _TP), lambda b, t: (b * nt + t, 0, 0)),
            pl.BlockSpec((1, _K, _TP), lambda b, t: (b * nt + t, 0, 0)),
        ],
        out_shape=[
            jax.ShapeDtypeStruct((_B * nt, _K, _TP), jnp.int32),
            jax.ShapeDtypeStruct((_B * nt, _K, _TP), jnp.int32),
        ],
    )(c8, c2.reshape(_B * nt, 1, _TP), pT8, p2.reshape(_B, 1, _N))

    col = jnp.transpose(colT.reshape(_B, nt, _K, _TP), (0, 1, 3, 2)).reshape(-1)
    sg = jnp.transpose(sgT.reshape(_B, nt, _K, _TP), (0, 1, 3, 2)).reshape(-1)

    new_h = _sc_gather_max(table, sg.reshape(_NW, _NCH, _ROWS))
    new_h = new_h.reshape(_B, _P, _OUT)

    row = jnp.repeat(jnp.arange(_B * _P, dtype=jnp.int32), _K)
    edge_index = jnp.stack([row, col], axis=0)
    centroids_batch = jnp.repeat(jnp.arange(_B, dtype=jnp.int32), _P)
    return (centroids, new_h, centroids_batch, edge_index)


# speedup vs baseline: 6.7904x; 6.7904x over previous
import functools

import jax
import jax.numpy as jnp
from jax import lax
from jax.experimental import pallas as pl
from jax.experimental.pallas import tpu as pltpu

_B = 4
_N = 8192
_P = 1024
_K = 32
_R2 = 0.2 ** 2
_OUT = 64
_TP = 128
_MROWS = 2048


_FS = 8
_FL = _N // _FS


def _red2(op, x):
    return op(op(x, axis=2, keepdims=True), axis=1, keepdims=True)


def _fps_body(px_ref, py_ref, pz_ref, cx_ref, cy_ref, cz_ref, dist_ref):
    px = px_ref[...]
    py = py_ref[...]
    pz = pz_ref[...]
    sub = lax.broadcasted_iota(jnp.int32, (_B, _FS, _FL), 1)
    lan = lax.broadcasted_iota(jnp.int32, (_B, _FS, _FL), 2)
    ion = sub * _FL + lan
    plane = lax.broadcasted_iota(jnp.int32, (_B, _P), 1)

    p0x = px[:, 0:1, 0:1]
    p0y = py[:, 0:1, 0:1]
    p0z = pz[:, 0:1, 0:1]
    dx = px - p0x
    dy = py - p0y
    dz = pz - p0z
    dist_ref[...] = (dx * dx + dy * dy) + dz * dz
    zmask = plane == 0
    cx_ref[...] = jnp.where(zmask, p0x[:, :, 0], 0.0)
    cy_ref[...] = jnp.where(zmask, p0y[:, :, 0], 0.0)
    cz_ref[...] = jnp.where(zmask, p0z[:, :, 0], 0.0)

    def body(i, _):
        dist = dist_ref[...]
        maxv = _red2(jnp.max, dist)
        idx = _red2(jnp.min, jnp.where(dist == maxv, ion, _N))
        oh = ion == idx
        cx = _red2(jnp.sum, jnp.where(oh, px, 0.0))
        cy = _red2(jnp.sum, jnp.where(oh, py, 0.0))
        cz = _red2(jnp.sum, jnp.where(oh, pz, 0.0))
        ddx = px - cx
        ddy = py - cy
        ddz = pz - cz
        d = (ddx * ddx + ddy * ddy) + ddz * ddz
        dist_ref[...] = jnp.minimum(dist, d)
        sm = plane == i
        cx_ref[...] += jnp.where(sm, cx[:, :, 0], 0.0)
        cy_ref[...] += jnp.where(sm, cy[:, :, 0], 0.0)
        cz_ref[...] += jnp.where(sm, cz[:, :, 0], 0.0)
        return 0

    lax.fori_loop(1, _P, body, 0)


def _mlp_body(x_ref, w1_ref, b1_ref, w2_ref, b2_ref, o_ref):
    i = pl.program_id(0)
    o_ref[...] = jnp.zeros_like(o_ref)

    @pl.when(i < _B * _N // _MROWS)
    def _():
        x = x_ref[...]
        a = jnp.dot(x, w1_ref[...], preferred_element_type=jnp.float32)
        a = jnp.maximum(a + b1_ref[...], 0.0)
        o = jnp.dot(a, w2_ref[...], preferred_element_type=jnp.float32)
        o_ref[:, :_OUT] = jnp.maximum(o + b2_ref[...], 0.0)


_SENT = _B * _N


def _select_body(c8_ref, c2_ref, pT_ref, p2_ref, col_ref, sg_ref):
    b = pl.program_id(0)
    c8 = c8_ref[0]
    pT = pT_ref[0]
    cp = jnp.dot(c8, pT, preferred_element_type=jnp.float32)
    d2 = (c2_ref[0, 0][:, None] + p2_ref[0, 0][None, :]) - 2.0 * cp
    inf = jnp.float32(jnp.inf)
    key = jnp.where(d2 <= _R2, d2, inf)
    lane = lax.broadcasted_iota(jnp.int32, (_TP, _N), 1)
    fallback = None
    for k in range(_K):
        minv = jnp.min(key, axis=1, keepdims=True)
        valid = minv[:, 0] < inf
        idx = jnp.min(jnp.where(key == minv, lane, _N), axis=1)
        g = idx + b * _N
        if k == 0:
            fallback = jnp.where(valid, g, _SENT)
        col_ref[0, k, :] = jnp.where(valid, g, -1)
        sg_ref[0, k, :] = jnp.where(valid, g, fallback)
        key = jnp.where(lane == idx[:, None], inf, key)


_NC = 2
_NS = 16
_NW = _NC * _NS
_CPW = _B * _P // _NW
_CCH = 4
_NCH = _CPW // _CCH
_ROWS = _CCH * _K


def _sc_gather_max(table, gidx3):
    from jax.experimental.pallas import tpu_sc as plsc

    mesh = plsc.VectorSubcoreMesh(core_axis_name="c", subcore_axis_name="s")

    @functools.partial(
        pl.kernel,
        mesh=mesh,
        out_type=jax.ShapeDtypeStruct((_B * _P, _OUT), jnp.float32),
        scratch_types=[
            pltpu.VMEM((_NCH, _ROWS), jnp.int32),
            pltpu.VMEM((_ROWS, 128), jnp.float32),
            pltpu.VMEM((_CPW, _OUT), jnp.float32),
            pltpu.SemaphoreType.DMA,
        ],
    )
    def k(table_hbm, gidx_hbm, out_hbm, idx_v, rows_v, out_v, sem):
        wid = lax.axis_index("s") * _NC + lax.axis_index("c")
        pltpu.sync_copy(gidx_hbm.at[wid], idx_v)

        def chunk(ci, _):
            pltpu.async_copy(table_hbm.at[idx_v.at[ci]], rows_v, sem).wait()

            def cent(j, _):
                base = j * _K
                for g in range(_OUT // 16):
                    sl = pl.ds(g * 16, 16)
                    acc = rows_v[base, sl]
                    for r in range(1, _K):
                        acc = jnp.maximum(acc, rows_v[base + r, sl])
                    out_v[ci * _CCH + j, sl] = acc
                return 0

            lax.fori_loop(0, _CCH, cent, 0)
            return 0

        lax.fori_loop(0, _NCH, chunk, 0)
        pltpu.sync_copy(out_v, out_hbm.at[pl.ds(wid * _CPW, _CPW)])

    return k(table, gidx3)


def kernel(pos, h, batch_indices, W1, b1, W2, b2):
    del batch_indices
    posB = pos.reshape(_B, _N, 3)
    px = posB[:, :, 0]
    py = posB[:, :, 1]
    pz = posB[:, :, 2]

    cx, cy, cz = pl.pallas_call(
        _fps_body,
        out_shape=[jax.ShapeDtypeStruct((_B, _P), jnp.float32)] * 3,
        scratch_shapes=[pltpu.VMEM((_B, _FS, _FL), jnp.float32)],
    )(px.reshape(_B, _FS, _FL), py.reshape(_B, _FS, _FL),
      pz.reshape(_B, _FS, _FL))
    centroids = jnp.stack([cx, cy, cz], axis=-1)

    hB = h.reshape(_B, _N, -1)
    feat = jnp.concatenate([hB, posB], axis=-1).reshape(_B * _N, 6)
    featp = jnp.concatenate(
        [feat, jnp.zeros((_B * _N, 2), jnp.float32)], axis=-1)
    W1p = jnp.concatenate([W1, jnp.zeros((2, _OUT), W1.dtype)], axis=0)
    nblk = _B * _N // _MROWS
    table = pl.pallas_call(
        _mlp_body,
        grid=(nblk + 1,),
        in_specs=[
            pl.BlockSpec((_MROWS, 8), lambda i: (jnp.minimum(i, nblk - 1), 0)),
            pl.BlockSpec((8, _OUT), lambda i: (0, 0)),
            pl.BlockSpec((1, _OUT), lambda i: (0, 0)),
            pl.BlockSpec((_OUT, _OUT), lambda i: (0, 0)),
            pl.BlockSpec((1, _OUT), lambda i: (0, 0)),
        ],
        out_specs=pl.BlockSpec((_MROWS, 128), lambda i: (i, 0)),
        out_shape=jax.ShapeDtypeStruct((_B * _N + _MROWS, 128), jnp.float32),
    )(featp, W1p, b1[None, :], W2, b2[None, :])

    c8 = jnp.concatenate(
        [centroids, jnp.zeros((_B, _P, 5), jnp.float32)], axis=-1)
    c2 = jnp.sum(centroids ** 2, -1)
    p2 = jnp.sum(posB ** 2, -1)
    pT = jnp.moveaxis(posB, 2, 1)
    pT8 = jnp.concatenate([pT, jnp.zeros((_B, 5, _N), jnp.float32)], axis=1)

    nt = _P // _TP
    colT, sgT = pl.pallas_call(
        _select_body,
        grid=(_B, nt),
        in_specs=[
            pl.BlockSpec((1, _TP, 8), lambda b, t: (b, t, 0)),
            pl.BlockSpec((1, 1, _TP), lambda b, t: (b * nt + t, 0, 0)),
            pl.BlockSpec((1, 8, _N), lambda b, t: (b, 0, 0)),
            pl.BlockSpec((1, 1, _N), lambda b, t: (b, 0, 0)),
        ],
        out_specs=[
            pl.BlockSpec((1, _K, _TP), lambda b, t: (b * nt + t, 0, 0)),
            pl.BlockSpec((1, _K, _TP), lambda b, t: (b * nt + t, 0, 0)),
        ],
        out_shape=[
            jax.ShapeDtypeStruct((_B * nt, _K, _TP), jnp.int32),
            jax.ShapeDtypeStruct((_B * nt, _K, _TP), jnp.int32),
        ],
    )(c8, c2.reshape(_B * nt, 1, _TP), pT8, p2.reshape(_B, 1, _N))

    col = jnp.transpose(colT.reshape(_B, nt, _K, _TP), (0, 1, 3, 2)).reshape(-1)
    sg = jnp.transpose(sgT.reshape(_B, nt, _K, _TP), (0, 1, 3, 2)).reshape(-1)

    new_h = _sc_gather_max(table, sg.reshape(_NW, _NCH, _ROWS))
    new_h = new_h.reshape(_B, _P, _OUT)

    row = jnp.repeat(jnp.arange(_B * _P, dtype=jnp.int32), _K)
    edge_index = jnp.stack([row, col], axis=0)
    centroids_batch = jnp.repeat(jnp.arange(_B, dtype=jnp.int32), _P)
    return (centroids, new_h, centroids_batch, edge_index)
